# Initial kernel scaffold; baseline (speedup 1.0000x reference)
import jax
import jax.numpy as jnp
from jax.experimental import pallas as pl


def kernel(xyz1, points1, xyz2, points2, t_emb, c_emb, mlp_w, mlp_b, gn0_g, gn0_b, conv1_w, conv1_b, gn1_g, gn1_b, conv2_w, conv2_b, gn2_g, gn2_b, tproj_w, tproj_b, cproj_w, cproj_b):
    raise NotImplementedError("write your pallas kernel here")


# speedup vs baseline: 814.4675x; 814.4675x over previous
import jax
import jax.numpy as jnp
from jax.experimental import pallas as pl
from jax.experimental.pallas import tpu as pltpu

_EPS_INTERP = 1e-8
_EPS_GN = 1e-5
_GROUPS = 8


def _dot(a, b, ca, cb):
    return jax.lax.dot_general(a, b, (((ca,), (cb,)), ((), ())),
                               preferred_element_type=jnp.float32)


def _tile_group_stats(x, groups):
    c = x.shape[0]
    ssum = jnp.sum(x, axis=1, keepdims=True)
    ssq = jnp.sum(x * x, axis=1, keepdims=True)
    gs = jnp.sum(ssum.reshape(groups, c // groups, 1), axis=1)
    gq = jnp.sum(ssq.reshape(groups, c // groups, 1), axis=1)
    return gs, gq


def _stage1_body(xyz1_ref, xyz2_ref, x1sq_ref, x2sq_ref, p1_ref, p2_ref,
                 w1_ref, w2_ref, b_ref, xp_ref, s_ref, q_ref):
    x1 = xyz1_ref[0]
    x2 = xyz2_ref[0]
    p1 = p1_ref[0]
    p2 = p2_ref[0]

    mm = _dot(x1, x2, 0, 0)
    dist = (-2.0 * mm + x1sq_ref[0]) + x2sq_ref[0]

    inf = jnp.float32(jnp.inf)
    d1 = jnp.min(dist, axis=1, keepdims=True)
    m1 = jnp.where(dist > d1, dist, inf)
    d2 = jnp.min(m1, axis=1, keepdims=True)
    m2 = jnp.where(m1 > d2, m1, inf)
    d3 = jnp.min(m2, axis=1, keepdims=True)

    sel = dist <= d3
    wr = jnp.where(sel, 1.0 / (dist + _EPS_INTERP), 0.0)
    norm = jnp.sum(wr, axis=1, keepdims=True)
    wm = wr / norm

    interp = _dot(p2, wm, 1, 1)
    xp = _dot(w1_ref[...], p1, 1, 0) + _dot(w2_ref[...], interp, 1, 0)
    xp = xp + b_ref[...]
    xp_ref[0] = xp

    gs, gq = _tile_group_stats(xp, _GROUPS)
    s_ref[0, 0] = gs
    q_ref[0, 0] = gq


def _stage2_body(xp_ref, m_ref, r_ref, g_ref, b_ref, w_ref, cb_ref,
                 xs_ref, h_ref, s_ref, q_ref):
    xp = xp_ref[0]
    xn = (xp - m_ref[0]) * r_ref[0] * g_ref[...] + b_ref[...]
    xs = xn * jax.nn.sigmoid(xn)
    xs_ref[0] = xs
    h = _dot(w_ref[...], xs, 1, 0) + cb_ref[...]
    h_ref[0] = h
    gs, gq = _tile_group_stats(h, _GROUPS)
    s_ref[0, 0] = gs
    q_ref[0, 0] = gq


def _stage3_body(h_ref, m_ref, r_ref, g_ref, b_ref, w_ref, cb_ref,
                 h2_ref, s_ref, q_ref):
    h = h_ref[0]
    hn = (h - m_ref[0]) * r_ref[0] * g_ref[...] + b_ref[...]
    hs = hn * jax.nn.sigmoid(hn)
    h2 = _dot(w_ref[...], hs, 1, 0) + cb_ref[...]
    h2_ref[0] = h2
    gs, gq = _tile_group_stats(h2, _GROUPS)
    s_ref[0, 0] = gs
    q_ref[0, 0] = gq


def _stage4_body(h2_ref, m_ref, r_ref, g_ref, b_ref, scale_ref, shift_ref,
                 xs_ref, out_ref):
    h2 = h2_ref[0]
    hn = (h2 - m_ref[0]) * r_ref[0] * g_ref[...] + b_ref[...]
    hn = hn * (1.0 + scale_ref[0]) + shift_ref[0]
    out_ref[0] = hn + xs_ref[0]


def _finalize(s, q, n, cpg):
    count = float(cpg * n)
    tot = jnp.sum(s, axis=1)[..., 0]
    totq = jnp.sum(q, axis=1)[..., 0]
    mean = tot / count
    var = totq / count - mean * mean
    rstd = jax.lax.rsqrt(var + _EPS_GN)
    mean_c = jnp.repeat(mean, cpg, axis=1)[:, :, None]
    rstd_c = jnp.repeat(rstd, cpg, axis=1)[:, :, None]
    return mean_c, rstd_c


def kernel(xyz1, points1, xyz2, points2, t_emb, c_emb, mlp_w, mlp_b, gn0_g,
           gn0_b, conv1_w, conv1_b, gn1_g, gn1_b, conv2_w, conv2_b, gn2_g,
           gn2_b, tproj_w, tproj_b, cproj_w, cproj_b):
    B, _, N = xyz1.shape
    S = xyz2.shape[2]
    C1 = points1.shape[1]
    C2 = points2.shape[1]
    OD = mlp_w.shape[0]
    ED = conv1_w.shape[0]
    G = _GROUPS

    TN = 256
    NT = N // TN

    f32 = jnp.float32
    w1 = mlp_w[:, :C1]
    w2 = mlp_w[:, C1:]
    col = lambda v: v.reshape(-1, 1)

    x1sq = jnp.sum(xyz1 ** 2, axis=1)[:, :, None]
    x2sq = jnp.sum(xyz2 ** 2, axis=1)[:, None, :]

    tp = t_emb @ tproj_w.T + tproj_b[None, :]
    cp = c_emb @ cproj_w.T + cproj_b[None, :]
    scale = (tp[:, :OD] + cp[:, :OD]).reshape(B, OD, 1)
    shift = (tp[:, OD:] + cp[:, OD:]).reshape(B, OD, 1)

    stats_shape = jax.ShapeDtypeStruct((B, NT, G, 1), f32)
    stats_spec = pl.BlockSpec((1, 1, G, 1), lambda b, n: (b, n, 0, 0))
    full2d = lambda a: pl.BlockSpec(a.shape, lambda b, n: (0, 0))
    tile_spec = lambda c: pl.BlockSpec((1, c, TN), lambda b, n: (b, 0, n))
    bcast_spec = lambda c, w: pl.BlockSpec((1, c, w), lambda b, n: (b, 0, 0))
    chan_spec = lambda c: pl.BlockSpec((1, c, 1), lambda b, n: (b, 0, 0))
    vec_spec = lambda c: pl.BlockSpec((c, 1), lambda b, n: (0, 0))

    grid = (B, NT)
    cparams = pltpu.CompilerParams(
        dimension_semantics=("parallel", "arbitrary"))

    xp, s0, q0 = pl.pallas_call(
        _stage1_body,
        grid=grid,
        in_specs=[
            tile_spec(3),
            bcast_spec(3, S),
            pl.BlockSpec((1, TN, 1), lambda b, n: (b, n, 0)),
            pl.BlockSpec((1, 1, S), lambda b, n: (b, 0, 0)),
            tile_spec(C1),
            bcast_spec(C2, S),
            full2d(w1), full2d(w2), vec_spec(OD),
        ],
        out_specs=[tile_spec(OD), stats_spec, stats_spec],
        out_shape=[jax.ShapeDtypeStruct((B, OD, N), f32),
                   stats_shape, stats_shape],
        compiler_params=cparams,
    )(xyz1, xyz2, x1sq, x2sq, points1, points2, w1, w2, col(mlp_b))

    m0, r0 = _finalize(s0, q0, N, OD // G)
    xs, h, s1, q1 = pl.pallas_call(
        _stage2_body,
        grid=grid,
        in_specs=[
            tile_spec(OD), chan_spec(OD), chan_spec(OD),
            vec_spec(OD), vec_spec(OD), full2d(conv1_w), vec_spec(ED),
        ],
        out_specs=[tile_spec(OD), tile_spec(ED), stats_spec, stats_spec],
        out_shape=[jax.ShapeDtypeStruct((B, OD, N), f32),
                   jax.ShapeDtypeStruct((B, ED, N), f32),
                   stats_shape, stats_shape],
        compiler_params=cparams,
    )(xp, m0, r0, col(gn0_g), col(gn0_b), conv1_w, col(conv1_b))

    m1, r1 = _finalize(s1, q1, N, ED // G)
    h2, s2, q2 = pl.pallas_call(
        _stage3_body,
        grid=grid,
        in_specs=[
            tile_spec(ED), chan_spec(ED), chan_spec(ED),
            vec_spec(ED), vec_spec(ED), full2d(conv2_w), vec_spec(OD),
        ],
        out_specs=[tile_spec(OD), stats_spec, stats_spec],
        out_shape=[jax.ShapeDtypeStruct((B, OD, N), f32),
                   stats_shape, stats_shape],
        compiler_params=cparams,
    )(h, m1, r1, col(gn1_g), col(gn1_b), conv2_w, col(conv2_b))

    m2, r2 = _finalize(s2, q2, N, OD // G)
    out = pl.pallas_call(
        _stage4_body,
        grid=grid,
        in_specs=[
            tile_spec(OD), chan_spec(OD), chan_spec(OD),
            vec_spec(OD), vec_spec(OD), chan_spec(OD), chan_spec(OD),
            tile_spec(OD),
        ],
        out_specs=tile_spec(OD),
        out_shape=jax.ShapeDtypeStruct((B, OD, N), f32),
        compiler_params=cparams,
    )(h2, m2, r2, col(gn2_g), col(gn2_b), scale, shift, xs)

    return out
